# baseline (speedup 1.0000x reference)
import functools

import jax
import jax.numpy as jnp
from jax import lax
from jax.experimental import pallas as pl
from jax.experimental.pallas import tpu as pltpu
from jax.experimental.pallas import tpu_sc as plsc

VOCAB_TILE = 256


@functools.lru_cache(maxsize=None)
def _make_sc_gather(V, D, B):
    info = plsc.get_sparse_core_info()
    NC, NS = info.num_cores, info.num_subcores
    NW = NC * NS
    assert B % (8 * NW) == 0
    b_per_w = B // NW
    mesh = plsc.VectorSubcoreMesh(core_axis_name="c", subcore_axis_name="s")

    @functools.partial(
        pl.kernel,
        mesh=mesh,
        out_type=jax.ShapeDtypeStruct((B, D), jnp.float32),
        scratch_types=[
            pltpu.VMEM((b_per_w,), jnp.int32),
            pltpu.VMEM((b_per_w, D), jnp.float32),
            pltpu.SemaphoreType.DMA,
        ],
        compiler_params=pltpu.CompilerParams(use_tc_tiling_on_sc=False),
    )
    def gather(table_hbm, idx_hbm, out_hbm, idx_v, rows_v, sem):
        wid = lax.axis_index("s") * NC + lax.axis_index("c")
        base = wid * b_per_w
        pltpu.sync_copy(idx_hbm.at[pl.ds(base, b_per_w)], idx_v)
        pltpu.async_copy(table_hbm.at[idx_v], rows_v, sem).wait()
        pltpu.sync_copy(rows_v, out_hbm.at[pl.ds(base, b_per_w)])

    return gather


def _mlp_body(flat_ref, W1_ref, b1_ref, W2_ref, b2_ref, out_ref, h_ref):
    @pl.when(pl.program_id(0) == 0)
    def _():
        h = jnp.maximum(
            jnp.dot(flat_ref[...], W1_ref[...],
                    preferred_element_type=jnp.float32) + b1_ref[...],
            0.0,
        )
        h_ref[...] = h.astype(jnp.bfloat16)

    out_ref[...] = jnp.dot(h_ref[...], W2_ref[...].astype(jnp.bfloat16),
                           preferred_element_type=jnp.float32) + b2_ref[...]


@functools.lru_cache(maxsize=None)
def _make_mlp(B, F, H, V):
    n_tiles = pl.cdiv(V, VOCAB_TILE)
    return pl.pallas_call(
        _mlp_body,
        grid=(n_tiles,),
        in_specs=[
            pl.BlockSpec((B, F), lambda i: (0, 0)),
            pl.BlockSpec((F, H), lambda i: (0, 0)),
            pl.BlockSpec((1, H), lambda i: (0, 0)),
            pl.BlockSpec((H, VOCAB_TILE), lambda i: (0, i)),
            pl.BlockSpec((1, VOCAB_TILE), lambda i: (0, i)),
        ],
        out_specs=pl.BlockSpec((B, VOCAB_TILE), lambda i: (0, i)),
        out_shape=jax.ShapeDtypeStruct((B, V), jnp.float32),
        scratch_shapes=[pltpu.VMEM((B, H), jnp.bfloat16)],
        compiler_params=pltpu.CompilerParams(
            dimension_semantics=("arbitrary",),
        ),
    )


def kernel(x, emb, W1, b1, W2, b2):
    B, BLOCK = x.shape
    V, D = emb.shape
    H = W1.shape[1]
    idx = x.reshape(-1).astype(jnp.int32)
    rows = _make_sc_gather(V, D, B * BLOCK)(emb, idx)
    flat = rows.reshape(B, BLOCK * D)
    mlp = _make_mlp(B, BLOCK * D, H, W2.shape[1])
    return mlp(flat, W1, b1.reshape(1, H), W2, b2.reshape(1, -1))


# speedup vs baseline: 3.6159x; 3.6159x over previous
import functools

import jax
import jax.numpy as jnp
from jax import lax
from jax.experimental import pallas as pl
from jax.experimental.pallas import tpu as pltpu
from jax.experimental.pallas import tpu_sc as plsc

VOCAB_TILE = 2048


@functools.lru_cache(maxsize=None)
def _make_sc_gather(V, D, B):
    info = plsc.get_sparse_core_info()
    NC, NS = info.num_cores, info.num_subcores
    NW = NC * NS
    assert B % (8 * NW) == 0
    b_per_w = B // NW
    mesh = plsc.VectorSubcoreMesh(core_axis_name="c", subcore_axis_name="s")

    @functools.partial(
        pl.kernel,
        mesh=mesh,
        out_type=jax.ShapeDtypeStruct((B, D), jnp.float32),
        scratch_types=[
            pltpu.VMEM((b_per_w,), jnp.int32),
            pltpu.VMEM((b_per_w, D), jnp.float32),
            pltpu.SemaphoreType.DMA,
        ],
        compiler_params=pltpu.CompilerParams(use_tc_tiling_on_sc=False),
    )
    def gather(table_hbm, idx_hbm, out_hbm, idx_v, rows_v, sem):
        wid = lax.axis_index("s") * NC + lax.axis_index("c")
        base = wid * b_per_w
        pltpu.sync_copy(idx_hbm.at[pl.ds(base, b_per_w)], idx_v)
        pltpu.async_copy(table_hbm.at[idx_v], rows_v, sem).wait()
        pltpu.sync_copy(rows_v, out_hbm.at[pl.ds(base, b_per_w)])

    return gather


def _mlp_body(flat_ref, W1_ref, b1_ref, W2t_ref, b2_ref, out_ref, ht_ref):
    @pl.when(pl.program_id(0) == 0)
    def _():
        ht = lax.dot_general(
            W1_ref[...], flat_ref[...],
            dimension_numbers=(((0,), (1,)), ((), ())),
            preferred_element_type=jnp.float32,
        )
        ht = jnp.maximum(ht + b1_ref[...], 0.0)
        ht_ref[...] = ht.astype(jnp.bfloat16)

    acc = jnp.dot(W2t_ref[...].astype(jnp.bfloat16), ht_ref[...],
                  preferred_element_type=jnp.float32)
    b2_col = lax.dot_general(
        b2_ref[...], jnp.ones((1, 1), jnp.float32),
        dimension_numbers=(((0,), (0,)), ((), ())),
        preferred_element_type=jnp.float32,
    )
    out_ref[...] = acc + b2_col


@functools.lru_cache(maxsize=None)
def _make_mlp(B, F, H, V):
    n_tiles = pl.cdiv(V, VOCAB_TILE)
    return pl.pallas_call(
        _mlp_body,
        grid=(n_tiles,),
        in_specs=[
            pl.BlockSpec((B, F), lambda i: (0, 0)),
            pl.BlockSpec((F, H), lambda i: (0, 0)),
            pl.BlockSpec((H, 1), lambda i: (0, 0)),
            pl.BlockSpec((VOCAB_TILE, H), lambda i: (i, 0)),
            pl.BlockSpec((1, VOCAB_TILE), lambda i: (0, i)),
        ],
        out_specs=pl.BlockSpec((VOCAB_TILE, B), lambda i: (i, 0)),
        out_shape=jax.ShapeDtypeStruct((V, B), jnp.float32),
        scratch_shapes=[pltpu.VMEM((H, B), jnp.bfloat16)],
        compiler_params=pltpu.CompilerParams(
            dimension_semantics=("arbitrary",),
        ),
    )


def kernel(x, emb, W1, b1, W2, b2):
    B, BLOCK = x.shape
    V, D = emb.shape
    H = W1.shape[1]
    VOC = W2.shape[1]
    idx = x.reshape(-1).astype(jnp.int32)
    rows = _make_sc_gather(V, D, B * BLOCK)(emb, idx)
    flat = rows.reshape(B, BLOCK * D)
    mlp = _make_mlp(B, BLOCK * D, H, VOC)
    out_t = mlp(flat, W1, b1.reshape(H, 1), W2.T, b2.reshape(1, VOC))
    return out_t.T


# speedup vs baseline: 3.6814x; 1.0181x over previous
import functools

import jax
import jax.numpy as jnp
from jax import lax
from jax.experimental import pallas as pl
from jax.experimental.pallas import tpu as pltpu
from jax.experimental.pallas import tpu_sc as plsc

VOCAB_TILE = 2048


@functools.lru_cache(maxsize=None)
def _make_sc_gather_t(D, V, N, BLOCK):
    info = plsc.get_sparse_core_info()
    NC, NS = info.num_cores, info.num_subcores
    NW = NC * NS
    assert D == NW
    BATCH = N // BLOCK
    mesh = plsc.VectorSubcoreMesh(core_axis_name="c", subcore_axis_name="s")

    @functools.partial(
        pl.kernel,
        mesh=mesh,
        out_type=jax.ShapeDtypeStruct((BLOCK * D, BATCH), jnp.float32),
        scratch_types=[
            pltpu.VMEM((N,), jnp.int32),
            pltpu.VMEM((N,), jnp.float32),
            pltpu.SemaphoreType.DMA,
        ],
        compiler_params=pltpu.CompilerParams(use_tc_tiling_on_sc=False),
    )
    def gather(table_hbm, idx_hbm, out_hbm, idx_v, col_v, sem):
        c = lax.axis_index("s") * NC + lax.axis_index("c")
        pltpu.sync_copy(idx_hbm, idx_v)
        row = table_hbm.at[c]
        pltpu.async_copy(row.at[idx_v], col_v, sem).wait()
        for j in range(BLOCK):
            pltpu.sync_copy(col_v.at[pl.ds(j * BATCH, BATCH)],
                            out_hbm.at[j * D + c])

    return gather


def _mlp_body(flatt_ref, W1_ref, b1_ref, W2t_ref, b2_ref, out_ref, ht_ref):
    @pl.when(pl.program_id(0) == 0)
    def _():
        ht = lax.dot_general(
            W1_ref[...], flatt_ref[...],
            dimension_numbers=(((0,), (0,)), ((), ())),
            preferred_element_type=jnp.float32,
        )
        ht = jnp.maximum(ht + b1_ref[...], 0.0)
        ht_ref[...] = ht.astype(jnp.bfloat16)

    acc = jnp.dot(W2t_ref[...].astype(jnp.bfloat16), ht_ref[...],
                  preferred_element_type=jnp.float32)
    b2_col = lax.dot_general(
        b2_ref[...], jnp.ones((1, 1), jnp.float32),
        dimension_numbers=(((0,), (0,)), ((), ())),
        preferred_element_type=jnp.float32,
    )
    out_ref[...] = acc + b2_col


@functools.lru_cache(maxsize=None)
def _make_mlp(B, F, H, V):
    n_tiles = pl.cdiv(V, VOCAB_TILE)
    return pl.pallas_call(
        _mlp_body,
        grid=(n_tiles,),
        in_specs=[
            pl.BlockSpec((F, B), lambda i: (0, 0)),
            pl.BlockSpec((F, H), lambda i: (0, 0)),
            pl.BlockSpec((H, 1), lambda i: (0, 0)),
            pl.BlockSpec((VOCAB_TILE, H), lambda i: (i, 0)),
            pl.BlockSpec((1, VOCAB_TILE), lambda i: (0, i)),
        ],
        out_specs=pl.BlockSpec((VOCAB_TILE, B), lambda i: (i, 0)),
        out_shape=jax.ShapeDtypeStruct((V, B), jnp.float32),
        scratch_shapes=[pltpu.VMEM((H, B), jnp.bfloat16)],
        compiler_params=pltpu.CompilerParams(
            dimension_semantics=("arbitrary",),
        ),
    )


def kernel(x, emb, W1, b1, W2, b2):
    B, BLOCK = x.shape
    V, D = emb.shape
    H = W1.shape[1]
    VOC = W2.shape[1]
    idx = x.T.reshape(-1).astype(jnp.int32)
    flat_t = _make_sc_gather_t(D, V, B * BLOCK, BLOCK)(emb.T, idx)
    mlp = _make_mlp(B, BLOCK * D, H, VOC)
    out_t = mlp(flat_t, W1, b1.reshape(H, 1), W2.T, b2.reshape(1, VOC))
    return out_t.T
